# baseline (speedup 1.0000x reference)
import jax
import jax.numpy as jnp
from jax import lax
from jax.experimental import pallas as pl
from jax.experimental.pallas import tpu as pltpu
from jax.experimental.pallas import tpu_sc as plsc

K = 1024
L = 16
UNROLL = 8
OCH = 4096
CAP1 = 2048
CAP2 = 256
U32 = jnp.uint32
I32 = jnp.int32


def _sc_topk_mask(n: int, k: int, rows_per_worker: int):
    nchunks = n // OCH

    def body(scores_hbm, out_hbm, xu0, xu1, oc0, oc1, hist, cum,
             cand1, cand2, sin0, sin1, sout0, sout1):
        wid = lax.axis_index("c") * 16 + lax.axis_index("s")
        row0 = wid * rows_per_worker
        lane = lax.iota(I32, L)
        lane_u = lax.iota(U32, L)
        ones = jnp.ones((L,), I32)
        zeros16 = jnp.zeros((L,), I32)
        one_f = jnp.full((L,), 1.0, jnp.float32)
        zero_f = jnp.zeros((L,), jnp.float32)
        xu = (xu0, xu1)
        oc = (oc0, oc1)
        sin = (sin0, sin1)
        sout = (sout0, sout1)

        @plsc.parallel_loop(0, 256 * L, L, unroll=UNROLL)
        def _(j):
            hist[pl.ds(j, L)] = zeros16

        def level_reduce(prefix, kk, shift):
            @plsc.parallel_loop(0, 256 * L, L, carry=zeros16)
            def acc_out(jj, acc):
                j = (256 * L - L) - jj
                acc = acc + hist[pl.ds(j, L)]
                cum[pl.ds(j, L)] = acc
                hist[pl.ds(j, L)] = zeros16
                return acc
            del acc_out

            def bstep(_, st):
                lo, hi = st
                mid = (lo + hi + 1) // 2
                tot = jnp.sum(cum[pl.ds(mid * L, L)])
                big = tot >= kk
                return (jnp.where(big, mid, lo), jnp.where(big, hi, mid - 1))

            bb, _ = lax.fori_loop(0, 8, bstep, (I32(0), I32(255)))
            above = jnp.where(
                bb < 255,
                jnp.sum(cum[pl.ds(jnp.minimum(bb + 1, I32(255)) * L, L)]),
                I32(0))
            prefix = prefix | (bb.astype(U32) << U32(shift))
            return prefix, kk - above

        in_copies = [None, None]
        in_copies[0] = pltpu.async_copy(scores_hbm.at[row0], xu[0], sin[0])
        out_copies = [None, None]

        for r in range(rows_per_worker):
            p = r % 2
            in_copies[p].wait()
            if r + 1 < rows_per_worker:
                in_copies[1 - p] = pltpu.async_copy(
                    scores_hbm.at[row0 + r + 1], xu[1 - p], sin[1 - p])
            xr = xu[p]

            @plsc.parallel_loop(0, n, L, unroll=UNROLL)
            def _(i):
                b = xr[pl.ds(i, L)]
                bi = plsc.bitcast(b, I32)
                flip = lax.shift_right_arithmetic(bi, I32(31)) | I32(
                    -2147483648)
                uk = plsc.bitcast(bi ^ flip, U32)
                idx = ((uk >> U32(20)) & U32(0xFF0)) | lane_u
                plsc.addupdate_scatter(hist, [plsc.bitcast(idx, I32)], ones)

            prefix, kk = level_reduce(U32(0), I32(k), 24)

            @plsc.parallel_loop(0, n, L, unroll=UNROLL, carry=zeros16)
            def cptr1(i, cp):
                bi = plsc.bitcast(xr[pl.ds(i, L)], I32)
                flip = lax.shift_right_arithmetic(bi, I32(31)) | I32(
                    -2147483648)
                uk = plsc.bitcast(bi ^ flip, U32)
                t = uk ^ prefix
                m = t < U32(1 << 24)
                idx = ((uk >> U32(12)) & U32(0xFF0)) | lane_u
                plsc.addupdate_scatter(hist, [plsc.bitcast(idx, I32)], ones,
                                       mask=m)
                cidx = (cp << I32(4)) | lane
                plsc.store_scatter(cand1, [cidx], plsc.bitcast(uk, I32),
                                   mask=m)
                return cp + m.astype(I32)

            prefix, kk = level_reduce(prefix, kk, 16)
            max1 = jnp.max(cptr1)

            @plsc.parallel_loop(0, max1 * L, L, unroll=4, carry=zeros16)
            def cptr2(i, cp):
                uk = plsc.bitcast(cand1[pl.ds(i, L)], U32)
                slotv = jnp.full((L,), 1, I32) * (i // L)
                t = uk ^ prefix
                m = (t < U32(1 << 16)) & (slotv < cptr1)
                idx = ((uk >> U32(4)) & U32(0xFF0)) | lane_u
                plsc.addupdate_scatter(hist, [plsc.bitcast(idx, I32)], ones,
                                       mask=m)
                slot = jnp.minimum(cp, I32(CAP2 - 1))
                cidx = (slot << I32(4)) | lane
                plsc.store_scatter(cand2, [cidx], plsc.bitcast(uk, I32),
                                   mask=m)
                return cp + m.astype(I32)

            prefix, kk = level_reduce(prefix, kk, 8)
            max2 = jnp.max(cptr2)
            safe2 = max2 <= I32(CAP2)

            @pl.when(safe2)
            def _():
                @plsc.parallel_loop(0, max2 * L, L, unroll=4)
                def _(i):
                    uk = plsc.bitcast(cand2[pl.ds(i, L)], U32)
                    slotv = jnp.full((L,), 1, I32) * (i // L)
                    t = uk ^ prefix
                    m = (t < U32(1 << 8)) & (slotv < cptr2)
                    idx = ((uk << U32(4)) & U32(0xFF0)) | lane_u
                    plsc.addupdate_scatter(hist, [plsc.bitcast(idx, I32)],
                                           ones, mask=m)

            @pl.when(jnp.logical_not(safe2))
            def _():
                @plsc.parallel_loop(0, n, L, unroll=UNROLL)
                def _(i):
                    bi = plsc.bitcast(xr[pl.ds(i, L)], I32)
                    flip = lax.shift_right_arithmetic(bi, I32(31)) | I32(
                        -2147483648)
                    uk = plsc.bitcast(bi ^ flip, U32)
                    t = uk ^ prefix
                    m = t < U32(1 << 8)
                    idx = ((uk << U32(4)) & U32(0xFF0)) | lane_u
                    plsc.addupdate_scatter(hist, [plsc.bitcast(idx, I32)],
                                           ones, mask=m)

            prefix, kk = level_reduce(prefix, kk, 0)

            pv = jnp.full((L,), 1, U32) * prefix
            ti = plsc.bitcast(pv, I32)
            unflip = (lax.shift_right_arithmetic(ti, I32(31)) ^ I32(-1)
                      ) | I32(-2147483648)
            thr_f = plsc.bitcast(ti ^ unflip, jnp.float32)

            for c in range(nchunks):
                q = c % 2
                if out_copies[q] is not None:
                    out_copies[q].wait()
                base = c * OCH
                ocq = oc[q]

                @plsc.parallel_loop(0, OCH, L, unroll=UNROLL)
                def _(i):
                    b = xr[pl.ds(base + i, L)]
                    ocq[pl.ds(i, L)] = jnp.where(b >= thr_f, one_f, zero_f)

                out_copies[q] = pltpu.async_copy(
                    ocq, out_hbm.at[row0 + r, pl.ds(base, OCH)], sout[q])

        out_copies[0].wait()
        out_copies[1].wait()

    return body


def kernel(scores):
    b, n = scores.shape
    k = min(K, n)
    info = plsc.get_sparse_core_info()
    nw = info.num_cores * info.num_subcores
    rpw = b // nw
    mesh = plsc.VectorSubcoreMesh(core_axis_name="c", subcore_axis_name="s")
    f = pl.kernel(
        _sc_topk_mask(n, k, rpw),
        out_type=jax.ShapeDtypeStruct((b, n), jnp.float32),
        mesh=mesh,
        compiler_params=pltpu.CompilerParams(needs_layout_passes=False),
        scratch_types=[
            pltpu.VMEM((n,), jnp.float32),
            pltpu.VMEM((n,), jnp.float32),
            pltpu.VMEM((OCH,), jnp.float32),
            pltpu.VMEM((OCH,), jnp.float32),
            pltpu.VMEM((256 * L,), I32),
            pltpu.VMEM((256 * L,), I32),
            pltpu.VMEM((CAP1 * L,), I32),
            pltpu.VMEM((CAP2 * L,), I32),
            pltpu.SemaphoreType.DMA,
            pltpu.SemaphoreType.DMA,
            pltpu.SemaphoreType.DMA,
            pltpu.SemaphoreType.DMA,
        ],
    )
    return f(scores)


# speedup vs baseline: 1.2257x; 1.2257x over previous
import jax
import jax.numpy as jnp
from jax import lax
from jax.experimental import pallas as pl
from jax.experimental.pallas import tpu as pltpu
from jax.experimental.pallas import tpu_sc as plsc

K = 1024
L = 16
UNROLL = 8
OCH = 4096
CAP1 = 2048
CAP2 = 256
U32 = jnp.uint32
I32 = jnp.int32


def _sc_topk_mask(n: int, k: int, rows_per_worker: int):
    nchunks = n // OCH

    def body(scores_hbm, out_hbm, xu0, xu1, oc0, oc1, hist, cum,
             cand1, cand2, sin0, sin1, sout0, sout1):
        wid = lax.axis_index("c") * 16 + lax.axis_index("s")
        row0 = wid * rows_per_worker
        lane = lax.iota(I32, L)
        lane_u = lax.iota(U32, L)
        ones = jnp.ones((L,), I32)
        zeros16 = jnp.zeros((L,), I32)
        one_f = jnp.full((L,), 1.0, jnp.float32)
        zero_f = jnp.zeros((L,), jnp.float32)
        xu = (xu0, xu1)
        oc = (oc0, oc1)
        sin = (sin0, sin1)
        sout = (sout0, sout1)

        @plsc.parallel_loop(0, 256 * L, L, unroll=UNROLL)
        def _(j):
            hist[pl.ds(j, L)] = zeros16

        def level_reduce(prefix, kk, shift):
            @plsc.parallel_loop(0, 256 * L, L, unroll=UNROLL, carry=zeros16)
            def acc_out(jj, acc):
                j = (256 * L - L) - jj
                acc = acc + hist[pl.ds(j, L)]
                cum[pl.ds(j, L)] = acc
                hist[pl.ds(j, L)] = zeros16
                return acc
            del acc_out

            def bstep(_, st):
                lo, hi = st
                mid = (lo + hi + 1) // 2
                tot = jnp.sum(cum[pl.ds(mid * L, L)])
                big = tot >= kk
                return (jnp.where(big, mid, lo), jnp.where(big, hi, mid - 1))

            bb, _ = lax.fori_loop(0, 8, bstep, (I32(0), I32(255)))
            above = jnp.where(
                bb < 255,
                jnp.sum(cum[pl.ds(jnp.minimum(bb + 1, I32(255)) * L, L)]),
                I32(0))
            prefix = prefix | (bb.astype(U32) << U32(shift))
            return prefix, kk - above

        in_copies = [None, None]
        in_copies[0] = pltpu.async_copy(scores_hbm.at[row0], xu[0], sin[0])
        out_copies = [None, None]

        for r in range(rows_per_worker):
            p = r % 2
            in_copies[p].wait()
            if r + 1 < rows_per_worker:
                in_copies[1 - p] = pltpu.async_copy(
                    scores_hbm.at[row0 + r + 1], xu[1 - p], sin[1 - p])
            xr = xu[p]

            @plsc.parallel_loop(0, n, L, unroll=UNROLL)
            def _(i):
                b = xr[pl.ds(i, L)]
                bi = plsc.bitcast(b, I32)
                flip = lax.shift_right_arithmetic(bi, I32(31)) | I32(
                    -2147483648)
                uk = plsc.bitcast(bi ^ flip, U32)
                xr[pl.ds(i, L)] = plsc.bitcast(uk, jnp.float32)
                idx = ((uk >> U32(20)) & U32(0xFF0)) | lane_u
                plsc.addupdate_scatter(hist, [plsc.bitcast(idx, I32)], ones)

            prefix, kk = level_reduce(U32(0), I32(k), 24)

            @plsc.parallel_loop(0, n, L, unroll=UNROLL, carry=zeros16)
            def cptr1(i, cp):
                uk = plsc.bitcast(xr[pl.ds(i, L)], U32)
                t = uk ^ prefix
                m = t < U32(1 << 24)
                idx = ((uk >> U32(12)) & U32(0xFF0)) | lane_u
                plsc.addupdate_scatter(hist, [plsc.bitcast(idx, I32)], ones,
                                       mask=m)
                cidx = (cp << I32(4)) | lane
                plsc.store_scatter(cand1, [cidx], plsc.bitcast(uk, I32),
                                   mask=m)
                return cp + m.astype(I32)

            prefix, kk = level_reduce(prefix, kk, 16)
            max1 = jnp.max(cptr1)

            @plsc.parallel_loop(0, max1 * L, L, unroll=4, carry=zeros16)
            def cptr2(i, cp):
                uk = plsc.bitcast(cand1[pl.ds(i, L)], U32)
                slotv = jnp.full((L,), 1, I32) * (i // L)
                t = uk ^ prefix
                m = (t < U32(1 << 16)) & (slotv < cptr1)
                idx = ((uk >> U32(4)) & U32(0xFF0)) | lane_u
                plsc.addupdate_scatter(hist, [plsc.bitcast(idx, I32)], ones,
                                       mask=m)
                slot = jnp.minimum(cp, I32(CAP2 - 1))
                cidx = (slot << I32(4)) | lane
                plsc.store_scatter(cand2, [cidx], plsc.bitcast(uk, I32),
                                   mask=m)
                return cp + m.astype(I32)

            prefix, kk = level_reduce(prefix, kk, 8)
            max2 = jnp.max(cptr2)
            safe2 = max2 <= I32(CAP2)

            @pl.when(safe2)
            def _():
                @plsc.parallel_loop(0, max2 * L, L, unroll=4)
                def _(i):
                    uk = plsc.bitcast(cand2[pl.ds(i, L)], U32)
                    slotv = jnp.full((L,), 1, I32) * (i // L)
                    t = uk ^ prefix
                    m = (t < U32(1 << 8)) & (slotv < cptr2)
                    idx = ((uk << U32(4)) & U32(0xFF0)) | lane_u
                    plsc.addupdate_scatter(hist, [plsc.bitcast(idx, I32)],
                                           ones, mask=m)

            @pl.when(jnp.logical_not(safe2))
            def _():
                @plsc.parallel_loop(0, n, L, unroll=UNROLL)
                def _(i):
                    uk = plsc.bitcast(xr[pl.ds(i, L)], U32)
                    t = uk ^ prefix
                    m = t < U32(1 << 8)
                    idx = ((uk << U32(4)) & U32(0xFF0)) | lane_u
                    plsc.addupdate_scatter(hist, [plsc.bitcast(idx, I32)],
                                           ones, mask=m)

            prefix, kk = level_reduce(prefix, kk, 0)

            for c in range(nchunks):
                q = c % 2
                if out_copies[q] is not None:
                    out_copies[q].wait()
                base = c * OCH
                ocq = oc[q]

                @plsc.parallel_loop(0, OCH, L, unroll=UNROLL)
                def _(i):
                    uk = plsc.bitcast(xr[pl.ds(base + i, L)], U32)
                    ocq[pl.ds(i, L)] = jnp.where(uk >= prefix, one_f, zero_f)

                out_copies[q] = pltpu.async_copy(
                    ocq, out_hbm.at[row0 + r, pl.ds(base, OCH)], sout[q])

        out_copies[0].wait()
        out_copies[1].wait()

    return body


def kernel(scores):
    b, n = scores.shape
    k = min(K, n)
    info = plsc.get_sparse_core_info()
    nw = info.num_cores * info.num_subcores
    rpw = b // nw
    mesh = plsc.VectorSubcoreMesh(core_axis_name="c", subcore_axis_name="s")
    f = pl.kernel(
        _sc_topk_mask(n, k, rpw),
        out_type=jax.ShapeDtypeStruct((b, n), jnp.float32),
        mesh=mesh,
        compiler_params=pltpu.CompilerParams(needs_layout_passes=False),
        scratch_types=[
            pltpu.VMEM((n,), jnp.float32),
            pltpu.VMEM((n,), jnp.float32),
            pltpu.VMEM((OCH,), jnp.float32),
            pltpu.VMEM((OCH,), jnp.float32),
            pltpu.VMEM((256 * L,), I32),
            pltpu.VMEM((256 * L,), I32),
            pltpu.VMEM((CAP1 * L,), I32),
            pltpu.VMEM((CAP2 * L,), I32),
            pltpu.SemaphoreType.DMA,
            pltpu.SemaphoreType.DMA,
            pltpu.SemaphoreType.DMA,
            pltpu.SemaphoreType.DMA,
        ],
    )
    return f(scores)
